# baseline (speedup 1.0000x reference)
import functools

import jax
import jax.numpy as jnp
import numpy as np
from jax import lax
from jax.experimental import pallas as pl
from jax.experimental.pallas import tpu as pltpu
from jax.experimental.pallas import tpu_sc as plsc

N_ITEMS = 100000
D = 32
B = 8
NPER = 512

_BN = 4096
_BNSH = 12
_NJ = -(-N_ITEMS // _BN)
_CAPW = 32
_WPG = 4
_CAP = _CAPW * _WPG
_NBKT = B * _NJ
_NSLOT = _NBKT * _CAP

_NC = 2
_NS = 16
_NW = _NC * _NS
_TOTAL = B * NPER
_CH = _TOTAL // _NW
_L = 16

_sc_mesh = plsc.VectorSubcoreMesh(core_axis_name="c", subcore_axis_name="s")

_MAGI = 0x4B000000
_MAGF = np.float32(8388608.0)


@functools.partial(
    pl.kernel,
    out_type=(
        jax.ShapeDtypeStruct((_NSLOT,), jnp.int32),
        jax.ShapeDtypeStruct((_NSLOT,), jnp.float32),
        jax.ShapeDtypeStruct((_NSLOT, D), jnp.float32),
    ),
    mesh=_sc_mesh,
    scratch_types=[
        pltpu.VMEM((_CH,), jnp.int32),
        pltpu.VMEM((_CH,), jnp.int32),
        pltpu.VMEM((_CH, D), jnp.float32),
        pltpu.VMEM((_CH,), jnp.float32),
        pltpu.VMEM((_L,), jnp.int32),
        pltpu.SemaphoreType.DMA,
    ],
    compiler_params=pltpu.CompilerParams(use_tc_tiling_on_sc=False),
)
def _sc_route(nodes_hbm, alpha_hbm, nout_hbm, nid_hbm, a_hbm, val_hbm,
              idx_v, dst_v, orows_v, alp_v, fill_v, sem):
    core = lax.axis_index("c")
    sub = lax.axis_index("s")
    wid = core * _NS + sub
    base = wid * _CH
    graph = wid // _WPG
    wrank = wid % _WPG

    share = _NSLOT // _NW
    fill_v[...] = jnp.full((_L,), -1, jnp.int32)

    def _fill(k, carry):
        pltpu.sync_copy(fill_v, nid_hbm.at[pl.ds(wid * share + k * _L, _L)])
        return carry

    lax.fori_loop(0, share // _L, _fill, 0)
    plsc.subcore_barrier()

    pltpu.sync_copy(nodes_hbm.at[pl.ds(base, _CH)], idx_v)
    acp = pltpu.async_copy(alpha_hbm.at[idx_v], alp_v, sem)
    ocp = pltpu.async_copy(nout_hbm.at[pl.ds(base, _CH)], orows_v, sem)
    acp.wait()
    ocp.wait()

    for c in range(_CH // _L):
        sl = pl.ds(c * _L, _L)
        a16 = alp_v[sl]
        for j in range(_L):
            r = c * _L + j
            aj = jnp.broadcast_to(lax.slice(a16, (j,), (j + 1,)), (_L,))
            orows_v[r, 0:_L] = aj * orows_v[r, 0:_L]
            orows_v[r, _L:D] = aj * orows_v[r, _L:D]

    def _tof(v):
        return lax.bitcast_convert_type(v + _MAGI, jnp.float32) - _MAGF

    def _toi(v):
        return lax.bitcast_convert_type(v + _MAGF, jnp.int32) - _MAGI

    lanes_f = _tof(lax.iota(jnp.int32, _L))
    for c in range(_CH // _L):
        sl = pl.ds(c * _L, _L)
        dst_v[sl] = lax.shift_right_logical(idx_v[sl], _BNSH) + graph * _NJ
    nbf_done = []
    for c in range(_CH // _L):
        sl = pl.ds(c * _L, _L)
        nbf = _tof(dst_v[sl])
        rankf = jnp.zeros((_L,), jnp.float32)
        for nbf_p in nbf_done:
            for k in range(_L):
                bk = jnp.broadcast_to(lax.slice(nbf_p, (k,), (k + 1,)), (_L,))
                rankf = rankf + jnp.maximum(0.0, 1.0 - jnp.abs(nbf - bk))
        for k in range(_L):
            bk = jnp.broadcast_to(lax.slice(nbf, (k,), (k + 1,)), (_L,))
            eqv = jnp.maximum(0.0, 1.0 - jnp.abs(nbf - bk))
            gt = jnp.minimum(1.0, jnp.maximum(0.0, lanes_f - k))
            rankf = rankf + eqv * gt
        dst_v[sl] = _toi(nbf * np.float32(_CAP) + rankf) + wrank * _CAPW
        nbf_done.append(nbf)
    ncp = pltpu.async_copy(idx_v, nid_hbm.at[dst_v], sem)
    acp2 = pltpu.async_copy(alp_v, a_hbm.at[dst_v], sem)
    vcp = pltpu.async_copy(orows_v, val_hbm.at[dst_v], sem)
    ncp.wait()
    acp2.wait()
    vcp.wait()


_MR = 40


def _fused_body(t_ref, nid_ref, a_ref, val_ref, o_ref):
    nid = nid_ref[0, 0]
    valid = nid >= 0
    n0 = pl.program_id(0) * _BN
    col = lax.broadcasted_iota(jnp.int32, (1, _BN), 1) + n0
    s = (nid.T == col).astype(jnp.bfloat16)
    m = jnp.concatenate(
        [val_ref[0, 0].T, a_ref[0, 0], jnp.zeros((_MR - D - 1, _CAP),
                                                 jnp.float32)], axis=0)
    m = jnp.where(valid, m, 0.0).astype(jnp.bfloat16)
    r = lax.dot_general(m, s, (((1,), (0,)), ((), ())),
                        preferred_element_type=jnp.float32)
    o_ref[0] = t_ref[...] * (1.0 - r[D:D + 1, :]) + r[:D, :]


_fused = pl.pallas_call(
    _fused_body,
    grid=(_NJ, B),
    in_specs=[
        pl.BlockSpec((D, _BN), lambda j, g: (0, j)),
        pl.BlockSpec((1, 1, 1, _CAP), lambda j, g: (g, j, 0, 0)),
        pl.BlockSpec((1, 1, 1, _CAP), lambda j, g: (g, j, 0, 0)),
        pl.BlockSpec((1, 1, _CAP, D), lambda j, g: (g, j, 0, 0)),
    ],
    out_specs=pl.BlockSpec((1, D, _BN), lambda j, g: (g, 0, j)),
    out_shape=jax.ShapeDtypeStruct((B, D, N_ITEMS), jnp.float32),
    compiler_params=pltpu.CompilerParams(
        dimension_semantics=("arbitrary", "arbitrary")),
)


def kernel(nodes_output, table, alpha, nums_nodes, nodes):
    del nums_nodes
    nid, a, val = _sc_route(nodes, alpha.reshape(N_ITEMS), nodes_output)
    dmaj = _fused(
        table.T,
        nid.reshape(B, _NJ, 1, _CAP),
        a.reshape(B, _NJ, 1, _CAP),
        val.reshape(B, _NJ, _CAP, D),
    )
    return dmaj.transpose(0, 2, 1)


# speedup vs baseline: 1.6402x; 1.6402x over previous
import functools

import jax
import jax.numpy as jnp
import numpy as np
from jax import lax
from jax.experimental import pallas as pl
from jax.experimental.pallas import tpu as pltpu
from jax.experimental.pallas import tpu_sc as plsc

N_ITEMS = 100000
D = 32
B = 8
NPER = 512

_BN = 8192
_BNSH = 13
_NJ = -(-N_ITEMS // _BN)
_CAPW = 48
_WPG = 4
_CAP = _CAPW * _WPG
_NBKT = B * _NJ
_NSLOT = _NBKT * _CAP

_NC = 2
_NS = 16
_NW = _NC * _NS
_TOTAL = B * NPER
_CH = _TOTAL // _NW
_L = 16

_sc_mesh = plsc.VectorSubcoreMesh(core_axis_name="c", subcore_axis_name="s")

_MAGI = 0x4B000000
_MAGF = np.float32(8388608.0)


@functools.partial(
    pl.kernel,
    out_type=(
        jax.ShapeDtypeStruct((_NSLOT,), jnp.int32),
        jax.ShapeDtypeStruct((_NSLOT,), jnp.float32),
        jax.ShapeDtypeStruct((_NSLOT, D), jnp.float32),
    ),
    mesh=_sc_mesh,
    scratch_types=[
        pltpu.VMEM((_CH,), jnp.int32),
        pltpu.VMEM((_CH,), jnp.int32),
        pltpu.VMEM((_CH, D), jnp.float32),
        pltpu.VMEM((_CH,), jnp.float32),
        pltpu.VMEM((_L,), jnp.int32),
        pltpu.SemaphoreType.DMA,
    ],
    compiler_params=pltpu.CompilerParams(use_tc_tiling_on_sc=False),
)
def _sc_route(nodes_hbm, alpha_hbm, nout_hbm, nid_hbm, a_hbm, val_hbm,
              idx_v, dst_v, orows_v, alp_v, fill_v, sem):
    core = lax.axis_index("c")
    sub = lax.axis_index("s")
    wid = core * _NS + sub
    base = wid * _CH
    graph = wid // _WPG
    wrank = wid % _WPG

    share = _NSLOT // _NW
    fill_v[...] = jnp.full((_L,), -1, jnp.int32)

    def _fill(k, carry):
        pltpu.sync_copy(fill_v, nid_hbm.at[pl.ds(wid * share + k * _L, _L)])
        return carry

    lax.fori_loop(0, share // _L, _fill, 0)
    plsc.subcore_barrier()

    pltpu.sync_copy(nodes_hbm.at[pl.ds(base, _CH)], idx_v)
    acp = pltpu.async_copy(alpha_hbm.at[idx_v], alp_v, sem)
    ocp = pltpu.async_copy(nout_hbm.at[pl.ds(base, _CH)], orows_v, sem)
    acp.wait()
    ocp.wait()

    for c in range(_CH // _L):
        sl = pl.ds(c * _L, _L)
        a16 = alp_v[sl]
        for j in range(_L):
            r = c * _L + j
            aj = jnp.broadcast_to(lax.slice(a16, (j,), (j + 1,)), (_L,))
            orows_v[r, 0:_L] = aj * orows_v[r, 0:_L]
            orows_v[r, _L:D] = aj * orows_v[r, _L:D]

    def _tof(v):
        return lax.bitcast_convert_type(v + _MAGI, jnp.float32) - _MAGF

    def _toi(v):
        return lax.bitcast_convert_type(v + _MAGF, jnp.int32) - _MAGI

    lanes_f = _tof(lax.iota(jnp.int32, _L))
    for c in range(_CH // _L):
        sl = pl.ds(c * _L, _L)
        dst_v[sl] = lax.shift_right_logical(idx_v[sl], _BNSH) + graph * _NJ
    nbf_done = []
    for c in range(_CH // _L):
        sl = pl.ds(c * _L, _L)
        nbf = _tof(dst_v[sl])
        rankf = jnp.zeros((_L,), jnp.float32)
        for nbf_p in nbf_done:
            for k in range(_L):
                bk = jnp.broadcast_to(lax.slice(nbf_p, (k,), (k + 1,)), (_L,))
                rankf = rankf + jnp.maximum(0.0, 1.0 - jnp.abs(nbf - bk))
        for k in range(_L):
            bk = jnp.broadcast_to(lax.slice(nbf, (k,), (k + 1,)), (_L,))
            eqv = jnp.maximum(0.0, 1.0 - jnp.abs(nbf - bk))
            gt = jnp.minimum(1.0, jnp.maximum(0.0, lanes_f - k))
            rankf = rankf + eqv * gt
        dst_v[sl] = _toi(nbf * np.float32(_CAP) + rankf) + wrank * _CAPW
        nbf_done.append(nbf)
    ncp = pltpu.async_copy(idx_v, nid_hbm.at[dst_v], sem)
    acp2 = pltpu.async_copy(alp_v, a_hbm.at[dst_v], sem)
    vcp = pltpu.async_copy(orows_v, val_hbm.at[dst_v], sem)
    ncp.wait()
    acp2.wait()
    vcp.wait()


_MR = 40


def _fused_body(t_ref, nid_ref, a_ref, val_ref, o_ref):
    t = t_ref[...]
    n0 = pl.program_id(0) * _BN
    col = lax.broadcasted_iota(jnp.int32, (1, _BN), 1) + n0
    for g in range(B):
        nid = nid_ref[g, 0]
        valid = nid >= 0
        s = (nid.T == col).astype(jnp.bfloat16)
        m = jnp.concatenate(
            [val_ref[g, 0].T, a_ref[g, 0], jnp.zeros((_MR - D - 1, _CAP),
                                                     jnp.float32)], axis=0)
        m = jnp.where(valid, m, 0.0).astype(jnp.bfloat16)
        r = lax.dot_general(m, s, (((1,), (0,)), ((), ())),
                            preferred_element_type=jnp.float32)
        o_ref[g] = t * (1.0 - r[D:D + 1, :]) + r[:D, :]


_fused = pl.pallas_call(
    _fused_body,
    grid=(_NJ,),
    in_specs=[
        pl.BlockSpec((D, _BN), lambda j: (0, j)),
        pl.BlockSpec((B, 1, 1, _CAP), lambda j: (0, j, 0, 0)),
        pl.BlockSpec((B, 1, 1, _CAP), lambda j: (0, j, 0, 0)),
        pl.BlockSpec((B, 1, _CAP, D), lambda j: (0, j, 0, 0)),
    ],
    out_specs=pl.BlockSpec((B, D, _BN), lambda j: (0, 0, j)),
    out_shape=jax.ShapeDtypeStruct((B, D, N_ITEMS), jnp.float32),
    compiler_params=pltpu.CompilerParams(
        dimension_semantics=("arbitrary",)),
)


def kernel(nodes_output, table, alpha, nums_nodes, nodes):
    del nums_nodes
    nid, a, val = _sc_route(nodes, alpha.reshape(N_ITEMS), nodes_output)
    dmaj = _fused(
        table.T,
        nid.reshape(B, _NJ, 1, _CAP),
        a.reshape(B, _NJ, 1, _CAP),
        val.reshape(B, _NJ, _CAP, D),
    )
    return dmaj.transpose(0, 2, 1)
